# Initial kernel scaffold; baseline (speedup 1.0000x reference)
import jax
import jax.numpy as jnp
from jax.experimental import pallas as pl


def kernel(image, flow):
    raise NotImplementedError("write your pallas kernel here")


# speedup vs baseline: 1.5367x; 1.5367x over previous
import functools

import jax
import jax.numpy as jnp
from jax import lax
from jax.experimental import pallas as pl
from jax.experimental.pallas import tpu as pltpu
from jax.experimental.pallas import tpu_sc as plsc

L = 16


def _div3_small(n):
    return (n * 43691) >> 17


def _div384(n):
    return _div3_small(n >> 7)


@functools.lru_cache(maxsize=None)
def _build_warp(B, H, W, C):
    N = B * H * W
    info = plsc.get_sparse_core_info()
    NC, NS = info.num_cores, info.num_subcores
    NW = NC * NS
    assert N % NW == 0
    P = N // NW
    NB = 128
    assert P % NB == 0
    NBLK = P // NB
    CG = C // L
    assert C % L == 0

    mesh = plsc.VectorSubcoreMesh(core_axis_name="c", subcore_axis_name="s")

    @functools.partial(
        pl.kernel,
        mesh=mesh,
        compiler_params=pltpu.CompilerParams(
            use_tc_tiling_on_sc=False, needs_layout_passes=False),
        out_type=jax.ShapeDtypeStruct((N, C), jnp.float32),
        scratch_types=[
            pltpu.VMEM((NB,), jnp.float32),
            pltpu.VMEM((NB,), jnp.float32),
            pltpu.VMEM((NB,), jnp.int32),
            pltpu.VMEM((NB,), jnp.int32),
            pltpu.VMEM((NB,), jnp.int32),
            pltpu.VMEM((NB,), jnp.int32),
            pltpu.VMEM((4 * NB,), jnp.float32),
            pltpu.VMEM((4 * NB, C), jnp.float32),
            pltpu.VMEM((NB, C), jnp.float32),
            pltpu.SemaphoreType.DMA,
        ],
    )
    def warp(img_hbm, fy_hbm, fx_hbm, out_hbm,
             fy_v, fx_v, idx_tl, idx_tr, idx_bl, idx_br, w_v, rows_v,
             out_v, sem):
        wid = lax.axis_index("s") * NC + lax.axis_index("c")
        base = wid * P

        def block(i, carry):
            s = base + i * NB
            pltpu.sync_copy(fy_hbm.at[pl.ds(s, NB)], fy_v)
            pltpu.sync_copy(fx_hbm.at[pl.ds(s, NB)], fx_v)

            def grp(j, carry2):
                pix = s + j * L + lax.iota(jnp.int32, L)
                q1 = _div384(pix)
                x = pix - q1 * W
                b = _div384(q1)
                y = q1 - b * H
                sl = pl.ds(j * L, L)
                qy = y.astype(jnp.float32) - fy_v[sl]
                qx = x.astype(jnp.float32) - fx_v[sl]
                qyc = jnp.clip(qy, 0.0, float(H - 2))
                qxc = jnp.clip(qx, 0.0, float(W - 2))
                y0 = qyc.astype(jnp.int32)
                x0 = qxc.astype(jnp.int32)
                ay = jnp.clip(qy - y0.astype(jnp.float32), 0.0, 1.0)
                ax = jnp.clip(qx - x0.astype(jnp.float32), 0.0, 1.0)
                r = b * (H * W) + y0 * W + x0
                idx_tl[sl] = r
                idx_tr[sl] = r + 1
                idx_bl[sl] = r + W
                idx_br[sl] = r + W + 1
                byc = 1.0 - ay
                bxc = 1.0 - ax
                w_v[sl] = byc * bxc
                w_v[pl.ds(NB + j * L, L)] = byc * ax
                w_v[pl.ds(2 * NB + j * L, L)] = ay * bxc
                w_v[pl.ds(3 * NB + j * L, L)] = ay * ax
                return carry2

            lax.fori_loop(0, NB // L, grp, 0)

            cps = [
                pltpu.async_copy(img_hbm.at[idx], rows_v.at[pl.ds(k * NB, NB)], sem)
                for k, idx in enumerate((idx_tl, idx_tr, idx_bl, idx_br))
            ]
            for cp in cps:
                cp.wait()

            def pix(j, carry2):
                jv = jnp.full((L,), j, jnp.int32)
                wtl = plsc.load_gather(w_v, [jv])
                wtr = plsc.load_gather(w_v, [jv + NB])
                wbl = plsc.load_gather(w_v, [jv + 2 * NB])
                wbr = plsc.load_gather(w_v, [jv + 3 * NB])
                for cg in range(CG):
                    csl = pl.ds(cg * L, L)
                    acc = wtl * rows_v[j, csl]
                    acc = acc + wtr * rows_v[NB + j, csl]
                    acc = acc + wbl * rows_v[2 * NB + j, csl]
                    acc = acc + wbr * rows_v[3 * NB + j, csl]
                    out_v[j, csl] = acc
                return carry2

            lax.fori_loop(0, NB, pix, 0)

            pltpu.sync_copy(out_v, out_hbm.at[pl.ds(s, NB)])
            return carry

        lax.fori_loop(0, NBLK, block, 0)

    return warp


def kernel(image, flow):
    B, H, W, C = image.shape
    img_flat = image.reshape(B * H * W, C)
    fy = flow[..., 0].reshape(-1)
    fx = flow[..., 1].reshape(-1)
    out = _build_warp(B, H, W, C)(img_flat, fy, fx)
    return out.reshape(B, H, W, C)


# speedup vs baseline: 2.0195x; 1.3141x over previous
import functools

import jax
import jax.numpy as jnp
from jax import lax
from jax.experimental import pallas as pl
from jax.experimental.pallas import tpu as pltpu
from jax.experimental.pallas import tpu_sc as plsc

L = 16


def _div3_small(n):
    return (n * 43691) >> 17


def _div384(n):
    return _div3_small(n >> 7)


@functools.lru_cache(maxsize=None)
def _build_warp(B, H, W, C):
    N = B * H * W
    info = plsc.get_sparse_core_info()
    NC, NS = info.num_cores, info.num_subcores
    NW = NC * NS
    assert N % NW == 0
    P = N // NW
    NB = 64
    assert P % (2 * NB) == 0
    NBLK = P // NB
    CG = C // L
    assert C % L == 0

    mesh = plsc.VectorSubcoreMesh(core_axis_name="c", subcore_axis_name="s")

    @functools.partial(
        pl.kernel,
        mesh=mesh,
        compiler_params=pltpu.CompilerParams(
            use_tc_tiling_on_sc=False, needs_layout_passes=False),
        out_type=jax.ShapeDtypeStruct((N, C), jnp.float32),
        scratch_types=[
            [pltpu.VMEM((NB,), jnp.float32) for _ in range(2)],
            [pltpu.VMEM((NB,), jnp.float32) for _ in range(2)],
            [[pltpu.VMEM((NB,), jnp.int32) for _ in range(4)]
             for _ in range(2)],
            [pltpu.VMEM((4 * NB,), jnp.float32) for _ in range(2)],
            [pltpu.VMEM((4 * NB, C), jnp.float32) for _ in range(2)],
            [pltpu.VMEM((NB, C), jnp.float32) for _ in range(2)],
            [pltpu.SemaphoreType.DMA for _ in range(2)],
            [pltpu.SemaphoreType.DMA for _ in range(2)],
            [pltpu.SemaphoreType.DMA for _ in range(2)],
        ],
    )
    def warp(img_hbm, fy_hbm, fx_hbm, out_hbm,
             fy, fx, idx, w, rows, out, sem_f, sem_g, sem_o):
        wid = lax.axis_index("s") * NC + lax.axis_index("c")
        base = wid * P

        def compute_idx(s, p):
            for jj in range(NB // L):
                pix = s + jj * L + lax.iota(jnp.int32, L)
                q1 = _div384(pix)
                xg = pix - q1 * W
                bg = _div384(q1)
                yg = q1 - bg * H
                sl = pl.ds(jj * L, L)
                qy = yg.astype(jnp.float32) - fy[p][sl]
                qx = xg.astype(jnp.float32) - fx[p][sl]
                qyc = jnp.clip(qy, 0.0, float(H - 2))
                qxc = jnp.clip(qx, 0.0, float(W - 2))
                y0 = qyc.astype(jnp.int32)
                x0 = qxc.astype(jnp.int32)
                ay = jnp.clip(qy - y0.astype(jnp.float32), 0.0, 1.0)
                ax = jnp.clip(qx - x0.astype(jnp.float32), 0.0, 1.0)
                r = bg * (H * W) + y0 * W + x0
                idx[p][0][sl] = r
                idx[p][1][sl] = r + 1
                idx[p][2][sl] = r + W
                idx[p][3][sl] = r + W + 1
                byc = 1.0 - ay
                bxc = 1.0 - ax
                w[p][sl] = byc * bxc
                w[p][pl.ds(NB + jj * L, L)] = byc * ax
                w[p][pl.ds(2 * NB + jj * L, L)] = ay * bxc
                w[p][pl.ds(3 * NB + jj * L, L)] = ay * ax

        def gather_copies(p):
            return [
                pltpu.make_async_copy(
                    img_hbm.at[idx[p][k]], rows[p].at[pl.ds(k * NB, NB)],
                    sem_g[p])
                for k in range(4)
            ]

        def fire_flow(s, p):
            pltpu.async_copy(fy_hbm.at[pl.ds(s, NB)], fy[p], sem_f[p])
            pltpu.async_copy(fx_hbm.at[pl.ds(s, NB)], fx[p], sem_f[p])

        def wait_flow(s, p):
            pltpu.make_async_copy(
                fy_hbm.at[pl.ds(s, NB)], fy[p], sem_f[p]).wait()
            pltpu.make_async_copy(
                fx_hbm.at[pl.ds(s, NB)], fx[p], sem_f[p]).wait()

        def blend(p):
            def pix(j, carry):
                jv = jnp.full((L,), j, jnp.int32)
                wtl = plsc.load_gather(w[p], [jv])
                wtr = plsc.load_gather(w[p], [jv + NB])
                wbl = plsc.load_gather(w[p], [jv + 2 * NB])
                wbr = plsc.load_gather(w[p], [jv + 3 * NB])
                for cg in range(CG):
                    csl = pl.ds(cg * L, L)
                    acc = wtl * rows[p][j, csl]
                    acc = acc + wtr * rows[p][NB + j, csl]
                    acc = acc + wbl * rows[p][2 * NB + j, csl]
                    acc = acc + wbr * rows[p][3 * NB + j, csl]
                    out[p][j, csl] = acc
                return carry

            lax.fori_loop(0, NB, pix, 0, unroll=2)

        def half(x, p):
            q = 1 - p
            s = base + x * NB

            @pl.when(x + 1 < NBLK)
            def _():
                wait_flow(s + NB, q)
                compute_idx(s + NB, q)
                for cp in gather_copies(q):
                    cp.start()

            @pl.when(x + 2 < NBLK)
            def _():
                fire_flow(s + 2 * NB, p)

            for cp in gather_copies(p):
                cp.wait()

            @pl.when(x >= 2)
            def _():
                pltpu.make_async_copy(
                    out[p], out_hbm.at[pl.ds(s, NB)], sem_o[p]).wait()

            blend(p)
            pltpu.async_copy(out[p], out_hbm.at[pl.ds(s, NB)], sem_o[p])

        pltpu.sync_copy(fy_hbm.at[pl.ds(base, NB)], fy[0])
        pltpu.sync_copy(fx_hbm.at[pl.ds(base, NB)], fx[0])
        compute_idx(base, 0)
        for cp0 in gather_copies(0):
            cp0.start()
        fire_flow(base + NB, 1)

        def pair(ii, carry):
            half(2 * ii, 0)
            half(2 * ii + 1, 1)
            return carry

        lax.fori_loop(0, NBLK // 2, pair, 0)

        for p_ in range(2):
            pltpu.make_async_copy(
                out[p_], out_hbm.at[pl.ds(base, NB)], sem_o[p_]).wait()

    return warp


def kernel(image, flow):
    B, H, W, C = image.shape
    img_flat = image.reshape(B * H * W, C)
    fy = flow[..., 0].reshape(-1)
    fx = flow[..., 1].reshape(-1)
    out = _build_warp(B, H, W, C)(img_flat, fy, fx)
    return out.reshape(B, H, W, C)


# speedup vs baseline: 2.5360x; 1.2558x over previous
import functools

import jax
import jax.numpy as jnp
from jax import lax
from jax.experimental import pallas as pl
from jax.experimental.pallas import tpu as pltpu
from jax.experimental.pallas import tpu_sc as plsc

L = 16


def _div3_small(n):
    return (n * 43691) >> 17


def _div384(n):
    return _div3_small(n >> 7)


@functools.lru_cache(maxsize=None)
def _build_warp(B, H, W, C):
    N = B * H * W
    info = plsc.get_sparse_core_info()
    NC, NS = info.num_cores, info.num_subcores
    NW = NC * NS
    assert N % NW == 0
    P = N // NW
    NB = 64
    assert P % (2 * NB) == 0
    NBLK = P // NB
    CG = C // L
    assert C % L == 0
    CP = 128

    mesh = plsc.VectorSubcoreMesh(core_axis_name="c", subcore_axis_name="s")

    @functools.partial(
        pl.kernel,
        mesh=mesh,
        compiler_params=pltpu.CompilerParams(
            use_tc_tiling_on_sc=True, needs_layout_passes=False),
        out_type=jax.ShapeDtypeStruct((N, CP), jnp.float32),
        scratch_types=[
            [pltpu.VMEM((NB,), jnp.float32) for _ in range(2)],
            [pltpu.VMEM((NB,), jnp.float32) for _ in range(2)],
            [[pltpu.VMEM((NB,), jnp.int32) for _ in range(4)]
             for _ in range(2)],
            [pltpu.VMEM((4 * NB,), jnp.float32) for _ in range(2)],
            [pltpu.VMEM((4 * NB, CP), jnp.float32) for _ in range(2)],
            [pltpu.VMEM((NB, CP), jnp.float32) for _ in range(2)],
            [pltpu.SemaphoreType.DMA for _ in range(2)],
            [pltpu.SemaphoreType.DMA for _ in range(2)],
            [pltpu.SemaphoreType.DMA for _ in range(2)],
        ],
    )
    def warp(img_hbm, fy_hbm, fx_hbm, out_hbm,
             fy, fx, idx, w, rows, out, sem_f, sem_g, sem_o):
        wid = lax.axis_index("s") * NC + lax.axis_index("c")
        base = wid * P

        def compute_idx(s, p):
            for jj in range(NB // L):
                pix = s + jj * L + lax.iota(jnp.int32, L)
                q1 = _div384(pix)
                xg = pix - q1 * W
                bg = _div384(q1)
                yg = q1 - bg * H
                sl = pl.ds(jj * L, L)
                qy = yg.astype(jnp.float32) - fy[p][sl]
                qx = xg.astype(jnp.float32) - fx[p][sl]
                qyc = jnp.clip(qy, 0.0, float(H - 2))
                qxc = jnp.clip(qx, 0.0, float(W - 2))
                y0 = qyc.astype(jnp.int32)
                x0 = qxc.astype(jnp.int32)
                ay = jnp.clip(qy - y0.astype(jnp.float32), 0.0, 1.0)
                ax = jnp.clip(qx - x0.astype(jnp.float32), 0.0, 1.0)
                r = bg * (H * W) + y0 * W + x0
                idx[p][0][sl] = r
                idx[p][1][sl] = r + 1
                idx[p][2][sl] = r + W
                idx[p][3][sl] = r + W + 1
                byc = 1.0 - ay
                bxc = 1.0 - ax
                w[p][sl] = byc * bxc
                w[p][pl.ds(NB + jj * L, L)] = byc * ax
                w[p][pl.ds(2 * NB + jj * L, L)] = ay * bxc
                w[p][pl.ds(3 * NB + jj * L, L)] = ay * ax

        def gather_copies(p):
            return [
                pltpu.make_async_copy(
                    img_hbm.at[idx[p][k]], rows[p].at[pl.ds(k * NB, NB)],
                    sem_g[p])
                for k in range(4)
            ]

        def fire_flow(s, p):
            pltpu.async_copy(fy_hbm.at[pl.ds(s, NB)], fy[p], sem_f[p])
            pltpu.async_copy(fx_hbm.at[pl.ds(s, NB)], fx[p], sem_f[p])

        def wait_flow(s, p):
            pltpu.make_async_copy(
                fy_hbm.at[pl.ds(s, NB)], fy[p], sem_f[p]).wait()
            pltpu.make_async_copy(
                fx_hbm.at[pl.ds(s, NB)], fx[p], sem_f[p]).wait()

        def blend(p):
            def pix(j, carry):
                jv = jnp.full((L,), j, jnp.int32)
                wtl = plsc.load_gather(w[p], [jv])
                wtr = plsc.load_gather(w[p], [jv + NB])
                wbl = plsc.load_gather(w[p], [jv + 2 * NB])
                wbr = plsc.load_gather(w[p], [jv + 3 * NB])
                for cg in range(CG):
                    csl = pl.ds(cg * L, L)
                    acc = wtl * rows[p][j, csl]
                    acc = acc + wtr * rows[p][NB + j, csl]
                    acc = acc + wbl * rows[p][2 * NB + j, csl]
                    acc = acc + wbr * rows[p][3 * NB + j, csl]
                    out[p][j, csl] = acc
                return carry

            lax.fori_loop(0, NB, pix, 0, unroll=2)

        def half(x, p):
            q = 1 - p
            s = base + x * NB

            @pl.when(x + 1 < NBLK)
            def _():
                wait_flow(s + NB, q)
                compute_idx(s + NB, q)
                for cp in gather_copies(q):
                    cp.start()

            @pl.when(x + 2 < NBLK)
            def _():
                fire_flow(s + 2 * NB, p)

            for cp in gather_copies(p):
                cp.wait()

            @pl.when(x >= 2)
            def _():
                pltpu.make_async_copy(
                    out[p], out_hbm.at[pl.ds(s, NB)], sem_o[p]).wait()

            blend(p)
            pltpu.async_copy(out[p], out_hbm.at[pl.ds(s, NB)], sem_o[p])

        pltpu.sync_copy(fy_hbm.at[pl.ds(base, NB)], fy[0])
        pltpu.sync_copy(fx_hbm.at[pl.ds(base, NB)], fx[0])
        compute_idx(base, 0)
        for cp0 in gather_copies(0):
            cp0.start()
        fire_flow(base + NB, 1)

        def pair(ii, carry):
            half(2 * ii, 0)
            half(2 * ii + 1, 1)
            return carry

        lax.fori_loop(0, NBLK // 2, pair, 0)

        for p_ in range(2):
            pltpu.make_async_copy(
                out[p_], out_hbm.at[pl.ds(base, NB)], sem_o[p_]).wait()

    return warp


def kernel(image, flow):
    B, H, W, C = image.shape
    img_pad = jnp.pad(image.reshape(B * H * W, C), ((0, 0), (0, 128 - C)))
    fy = flow[..., 0].reshape(-1)
    fx = flow[..., 1].reshape(-1)
    out = _build_warp(B, H, W, C)(img_pad, fy, fx)
    return out[:, :C].reshape(B, H, W, C)
